# baseline (speedup 1.0000x reference)
import jax
import jax.numpy as jnp
from jax import lax
from jax.experimental import pallas as pl
from jax.experimental.pallas import tpu as pltpu
from jax.experimental.pallas import tpu_sc as plsc

_B = 1024
_A = 50001
_V = (_A - 1) // 5
_NC = 2
_NS = 16
_NW = _NC * _NS
_RPT = _B // _NW
_NBUF = 3


def _row_stats_body(idx_ref, val_ref, j0_ref, neg_ref, sv_ref):
    idx = idx_ref[:, :]
    v = val_ref[:, :]
    hit = idx < (_A - 1)
    j0 = lax.rem(idx, _V)
    m = jnp.maximum(v, 0.0)
    lse_hit = m + jnp.log((_V - 1) * jnp.exp(-m) + jnp.exp(v - m))
    lse = jnp.where(hit, lse_hit, jnp.log(jnp.float32(_V)))
    neg = -lse
    j0_ref[:, :] = j0
    neg_ref[:, :] = neg
    sv_ref[:, :] = jnp.where(hit, v - lse, neg)


def _row_stats(idx2, val2):
    return pl.pallas_call(
        _row_stats_body,
        out_shape=[
            jax.ShapeDtypeStruct((_B, 1), jnp.int32),
            jax.ShapeDtypeStruct((_B, 1), jnp.float32),
            jax.ShapeDtypeStruct((_B, 1), jnp.float32),
        ],
    )(idx2, val2)


def _sc_fill(j0_h, neg_h, sv_h, out_h, j0_v, neg_v, sv_v, bufs, sems):
    wid = lax.axis_index("s") * _NC + lax.axis_index("c")
    base = wid * _RPT
    pltpu.sync_copy(j0_h.at[pl.ds(base, _RPT)], j0_v)
    pltpu.sync_copy(neg_h.at[pl.ds(base, _RPT)], neg_v)
    pltpu.sync_copy(sv_h.at[pl.ds(base, _RPT)], sv_v)
    lanes = lax.iota(jnp.int32, 16)
    handles = []

    for k in range(_RPT):
        g, r = divmod(k, 16)
        if r == 0:
            neg16 = neg_v[pl.ds(g * 16, 16)]
            sv16 = sv_v[pl.ds(g * 16, 16)]
            j016 = j0_v[pl.ds(g * 16, 16)]
        neg_s = neg16[r]
        sv_s = sv16[r]
        j0_s = j016[r]
        negs = jnp.full((16,), neg_s)
        buf = bufs[k % _NBUF]
        if k >= _NBUF:
            handles[k - _NBUF].wait()

        def _fill(i, negs, buf=buf):
            buf[pl.ds(pl.multiple_of(i * 16, 16), 16)] = negs
            return negs

        lax.fori_loop(0, _V // 16, _fill, negs)
        lane = lax.rem(j0_s, 16)
        start = j0_s - lane
        merged = jnp.where(lanes == lane, jnp.full((16,), sv_s), negs)
        buf[pl.ds(pl.multiple_of(start, 16), 16)] = merged
        handles.append(pltpu.async_copy(buf, out_h.at[base + k], sems[k % _NBUF]))
    for k in range(_NBUF):
        handles[_RPT - _NBUF + k].wait()


@jax.jit
def _run(actions_idx, val):
    idx2 = actions_idx.reshape(_B, 1).astype(jnp.int32)
    val2 = val.reshape(_B, 1).astype(jnp.float32)
    j0, neg, sv = _row_stats(idx2, val2)
    mesh = plsc.VectorSubcoreMesh(core_axis_name="c", subcore_axis_name="s")
    fill = pl.kernel(
        _sc_fill,
        out_type=jax.ShapeDtypeStruct((_B, _V), jnp.float32),
        mesh=mesh,
        scratch_types=[
            pltpu.VMEM((_RPT,), jnp.int32),
            pltpu.VMEM((_RPT,), jnp.float32),
            pltpu.VMEM((_RPT,), jnp.float32),
            [pltpu.VMEM((_V,), jnp.float32) for _ in range(_NBUF)],
            [pltpu.SemaphoreType.DMA for _ in range(_NBUF)],
        ],
    )
    return fill(j0.reshape(_B), neg.reshape(_B), sv.reshape(_B))


def kernel(actions_idx, action_space_dim, val):
    del action_space_dim
    return _run(actions_idx, val)


# speedup vs baseline: 1.4055x; 1.4055x over previous
import jax
import jax.numpy as jnp
from jax import lax
from jax.experimental import pallas as pl
from jax.experimental.pallas import tpu as pltpu
from jax.experimental.pallas import tpu_sc as plsc

_B = 1024
_A = 50001
_V = (_A - 1) // 5
_NC = 2
_NS = 16
_NW = _NC * _NS
_RPT = _B // _NW
_RPB = 4
_NBUF = 2


def _row_stats_body(idx_ref, val_ref, j0_ref, neg_ref, sv_ref):
    idx = idx_ref[:, :]
    v = val_ref[:, :]
    hit = idx < (_A - 1)
    j0 = lax.rem(idx, _V)
    m = jnp.maximum(v, 0.0)
    lse_hit = m + jnp.log((_V - 1) * jnp.exp(-m) + jnp.exp(v - m))
    lse = jnp.where(hit, lse_hit, jnp.log(jnp.float32(_V)))
    neg = -lse
    j0_ref[:, :] = j0
    neg_ref[:, :] = neg
    sv_ref[:, :] = jnp.where(hit, v - lse, neg)


def _row_stats(idx2, val2):
    return pl.pallas_call(
        _row_stats_body,
        out_shape=[
            jax.ShapeDtypeStruct((_B, 1), jnp.int32),
            jax.ShapeDtypeStruct((_B, 1), jnp.float32),
            jax.ShapeDtypeStruct((_B, 1), jnp.float32),
        ],
    )(idx2, val2)


def _sc_fill(j0_h, neg_h, sv_h, out_h, j0_v, neg_v, sv_v, bufs, sems):
    wid = lax.axis_index("s") * _NC + lax.axis_index("c")
    base = wid * _RPT
    pltpu.sync_copy(j0_h.at[pl.ds(base, _RPT)], j0_v)
    pltpu.sync_copy(neg_h.at[pl.ds(base, _RPT)], neg_v)
    pltpu.sync_copy(sv_h.at[pl.ds(base, _RPT)], sv_v)
    lanes = lax.iota(jnp.int32, 16)
    handles = []
    neg16 = sv16 = j016 = None

    for c in range(_RPT // _RPB):
        buf = bufs[c % _NBUF]
        if c >= _NBUF:
            handles[c - _NBUF].wait()
        for ri in range(_RPB):
            k = c * _RPB + ri
            g, r = divmod(k, 16)
            if r == 0:
                neg16 = neg_v[pl.ds(g * 16, 16)]
                sv16 = sv_v[pl.ds(g * 16, 16)]
                j016 = j0_v[pl.ds(g * 16, 16)]
            negs = jnp.full((16,), neg16[r])

            @plsc.parallel_loop(ri * _V, (ri + 1) * _V, 16, unroll=25)
            def _fill(i, negs=negs, buf=buf):
                buf[pl.ds(pl.multiple_of(i, 16), 16)] = negs

            j0_s = j016[r]
            lane = lax.rem(j0_s, 16)
            start = ri * _V + j0_s - lane
            merged = jnp.where(lanes == lane, jnp.full((16,), sv16[r]), negs)
            buf[pl.ds(pl.multiple_of(start, 16), 16)] = merged
        handles.append(pltpu.async_copy(
            buf, out_h.at[pl.ds((base + c * _RPB) * _V, _RPB * _V)],
            sems[c % _NBUF]))
    for c in range(_NBUF):
        handles[_RPT // _RPB - _NBUF + c].wait()


@jax.jit
def _run(actions_idx, val):
    idx2 = actions_idx.reshape(_B, 1).astype(jnp.int32)
    val2 = val.reshape(_B, 1).astype(jnp.float32)
    j0, neg, sv = _row_stats(idx2, val2)
    mesh = plsc.VectorSubcoreMesh(core_axis_name="c", subcore_axis_name="s")
    fill = pl.kernel(
        _sc_fill,
        out_type=jax.ShapeDtypeStruct((_B * _V,), jnp.float32),
        mesh=mesh,
        scratch_types=[
            pltpu.VMEM((_RPT,), jnp.int32),
            pltpu.VMEM((_RPT,), jnp.float32),
            pltpu.VMEM((_RPT,), jnp.float32),
            [pltpu.VMEM((_RPB * _V,), jnp.float32) for _ in range(_NBUF)],
            [pltpu.SemaphoreType.DMA for _ in range(_NBUF)],
        ],
    )
    out = fill(j0.reshape(_B), neg.reshape(_B), sv.reshape(_B))
    return out.reshape(_B, _V)


def kernel(actions_idx, action_space_dim, val):
    del action_space_dim
    return _run(actions_idx, val)


# speedup vs baseline: 1.6463x; 1.1714x over previous
import jax
import jax.numpy as jnp
from jax import lax
from jax.experimental import pallas as pl
from jax.experimental.pallas import tpu as pltpu
from jax.experimental.pallas import tpu_sc as plsc

_B = 1024
_A = 50001
_V = (_A - 1) // 5
_NC = 2
_NS = 16
_NW = _NC * _NS
_RPT = _B // _NW
_RPB = 4
_NBUF = 2


def _row_stats_body(idx_ref, val_ref, j0_ref, neg_ref, sv_ref):
    idx = idx_ref[:, :]
    v = val_ref[:, :]
    hit = idx < (_A - 1)
    j0 = lax.rem(idx, _V)
    m = jnp.maximum(v, 0.0)
    lse_hit = m + jnp.log((_V - 1) * jnp.exp(-m) + jnp.exp(v - m))
    lse = jnp.where(hit, lse_hit, jnp.log(jnp.float32(_V)))
    neg = -lse
    j0_ref[:, :] = j0
    neg_ref[:, :] = neg
    sv_ref[:, :] = jnp.where(hit, v - lse, neg)


def _row_stats(idx2, val2):
    return pl.pallas_call(
        _row_stats_body,
        out_shape=[
            jax.ShapeDtypeStruct((_B, 1), jnp.int32),
            jax.ShapeDtypeStruct((_B, 1), jnp.float32),
            jax.ShapeDtypeStruct((_B, 1), jnp.float32),
        ],
    )(idx2, val2)


def _sc_fill(j0_h, neg_h, sv_h, out_h, j0_v, neg_v, sv_v, bufs, sems):
    wid = lax.axis_index("s") * _NC + lax.axis_index("c")
    base = wid * _RPT
    pltpu.sync_copy(j0_h.at[pl.ds(base, _RPT)], j0_v)
    pltpu.sync_copy(neg_h.at[pl.ds(base, _RPT)], neg_v)
    pltpu.sync_copy(sv_h.at[pl.ds(base, _RPT)], sv_v)
    lanes = lax.iota(jnp.int32, 16)
    handles = []
    neg16 = sv16 = j016 = None

    for c in range(_RPT // _RPB):
        buf = bufs[c % _NBUF]
        if c >= _NBUF:
            handles[c - _NBUF].wait()
        for ri in range(_RPB):
            k = c * _RPB + ri
            g, r = divmod(k, 16)
            if r == 0:
                neg16 = neg_v[pl.ds(g * 16, 16)]
                sv16 = sv_v[pl.ds(g * 16, 16)]
                j016 = j0_v[pl.ds(g * 16, 16)]
            negs = jnp.full((16,), neg16[r])

            @plsc.parallel_loop(0, _V, 16, unroll=25)
            def _fill(i, negs=negs, buf=buf, ri=ri):
                buf[ri, pl.ds(pl.multiple_of(i, 16), 16)] = negs

            j0_s = j016[r]
            lane = lax.rem(j0_s, 16)
            start = j0_s - lane
            merged = jnp.where(lanes == lane, jnp.full((16,), sv16[r]), negs)
            buf[ri, pl.ds(pl.multiple_of(start, 16), 16)] = merged
        handles.append(pltpu.async_copy(
            buf, out_h.at[pl.ds(base + c * _RPB, _RPB)],
            sems[c % _NBUF]))
    for c in range(_NBUF):
        handles[_RPT // _RPB - _NBUF + c].wait()


@jax.jit
def _run(actions_idx, val):
    idx2 = actions_idx.reshape(_B, 1).astype(jnp.int32)
    val2 = val.reshape(_B, 1).astype(jnp.float32)
    j0, neg, sv = _row_stats(idx2, val2)
    mesh = plsc.VectorSubcoreMesh(core_axis_name="c", subcore_axis_name="s")
    fill = pl.kernel(
        _sc_fill,
        out_type=jax.ShapeDtypeStruct((_B, _V), jnp.float32),
        mesh=mesh,
        compiler_params=pltpu.CompilerParams(use_tc_tiling_on_sc=True),
        scratch_types=[
            pltpu.VMEM((_RPT,), jnp.int32),
            pltpu.VMEM((_RPT,), jnp.float32),
            pltpu.VMEM((_RPT,), jnp.float32),
            [pltpu.VMEM((_RPB, _V), jnp.float32) for _ in range(_NBUF)],
            [pltpu.SemaphoreType.DMA for _ in range(_NBUF)],
        ],
    )
    return fill(j0.reshape(_B), neg.reshape(_B), sv.reshape(_B))


def kernel(actions_idx, action_space_dim, val):
    del action_space_dim
    return _run(actions_idx, val)


# speedup vs baseline: 2.8252x; 1.7160x over previous
import jax
import jax.numpy as jnp
from jax import lax
from jax.experimental import pallas as pl
from jax.experimental.pallas import tpu as pltpu

_B = 1024
_A = 50001
_V = (_A - 1) // 5
_RB = 128
_NB = 4


def _fill_body(idx_ref, val_ref, out_hbm, *scratch):
    bufs = scratch[:_NB]
    sems = scratch[_NB:]
    idx = idx_ref[:, :]
    v = val_ref[:, :]
    hit = idx < (_A - 1)
    j0 = jnp.where(hit, lax.rem(idx, _V), -1)
    m = jnp.maximum(v, 0.0)
    lse_hit = m + jnp.log((_V - 1) * jnp.exp(-m) + jnp.exp(v - m))
    lse = jnp.where(hit, lse_hit, jnp.log(jnp.float32(_V)))
    pos = v - lse
    neg = -lse
    cols = lax.broadcasted_iota(jnp.int32, (_RB, _V), 1)
    copies = []
    for s in range(_B // _RB):
        buf = bufs[s % _NB]
        sem = sems[s % _NB]
        if s >= _NB:
            copies[s - _NB].wait()
        r0 = s * _RB
        buf[:, :] = jnp.where(cols == j0[r0:r0 + _RB],
                              pos[r0:r0 + _RB], neg[r0:r0 + _RB])
        cp = pltpu.make_async_copy(buf, out_hbm.at[pl.ds(r0, _RB)], sem)
        cp.start()
        copies.append(cp)
    for s in range(_NB):
        copies[_B // _RB - _NB + s].wait()


@jax.jit
def _run(actions_idx, val):
    idx2 = actions_idx.reshape(_B, 1).astype(jnp.int32)
    val2 = val.reshape(_B, 1).astype(jnp.float32)
    return pl.pallas_call(
        _fill_body,
        in_specs=[
            pl.BlockSpec(memory_space=pltpu.VMEM),
            pl.BlockSpec(memory_space=pltpu.VMEM),
        ],
        out_specs=pl.BlockSpec(memory_space=pltpu.MemorySpace.HBM),
        out_shape=jax.ShapeDtypeStruct((_B, _V), jnp.float32),
        scratch_shapes=(
            [pltpu.VMEM((_RB, _V), jnp.float32) for _ in range(_NB)]
            + [pltpu.SemaphoreType.DMA for _ in range(_NB)]
        ),
    )(idx2, val2)


def kernel(actions_idx, action_space_dim, val):
    del action_space_dim
    return _run(actions_idx, val)


# speedup vs baseline: 2.8766x; 1.0182x over previous
import functools

import jax
import jax.numpy as jnp
from jax.experimental import pallas as pl

_B = 1024
_A = 50001
_V = (_A - 1) // 5
_ROWS = 128


def _fill_body(idx_ref, val_ref, out_ref):
    idx = idx_ref[:, :]
    v = val_ref[:, :]
    hit = idx < (_A - 1)
    j0 = jnp.where(hit, jax.lax.rem(idx, _V), -1)
    m = jnp.maximum(v, 0.0)
    lse_hit = m + jnp.log((_V - 1) * jnp.exp(-m) + jnp.exp(v - m))
    lse = jnp.where(hit, lse_hit, jnp.log(jnp.float32(_V)))
    pos = v - lse
    neg = -lse
    cols = jax.lax.broadcasted_iota(jnp.int32, (_ROWS, _V), 1)
    out_ref[:, :] = jnp.where(cols == j0, pos, neg)


@jax.jit
def _run(actions_idx, val):
    idx2 = actions_idx.reshape(_B, 1).astype(jnp.int32)
    val2 = val.reshape(_B, 1).astype(jnp.float32)
    grid = (_B // _ROWS,)
    return pl.pallas_call(
        _fill_body,
        grid=grid,
        in_specs=[
            pl.BlockSpec((_ROWS, 1), lambda i: (i, 0)),
            pl.BlockSpec((_ROWS, 1), lambda i: (i, 0)),
        ],
        out_specs=pl.BlockSpec((_ROWS, _V), lambda i: (i, 0)),
        out_shape=jax.ShapeDtypeStruct((_B, _V), jnp.float32),
    )(idx2, val2)


def kernel(actions_idx, action_space_dim, val):
    del action_space_dim
    return _run(actions_idx, val)


# speedup vs baseline: 9.6024x; 3.3381x over previous
import jax
import jax.numpy as jnp
from jax import lax
from jax.experimental import pallas as pl

_B = 1024
_A = 50001
_V = (_A - 1) // 5
_JB = 1000


def _fill_body(idx_ref, val_ref, out_ref):
    idx = idx_ref[0:1, :]
    v = val_ref[0:1, :]
    hit = idx < (_A - 1)
    j0 = jnp.where(hit, lax.rem(idx, _V), -1)
    m = jnp.maximum(v, 0.0)
    lse_hit = m + jnp.log((_V - 1) * jnp.exp(-m) + jnp.exp(v - m))
    lse = jnp.where(hit, lse_hit, jnp.log(jnp.float32(_V)))
    pos = v - lse
    neg = -lse
    jrow = (lax.broadcasted_iota(jnp.int32, (_JB, _B), 0)
            + pl.program_id(0) * _JB)
    out_ref[:, :] = jnp.where(jrow == j0, pos, neg)


@jax.jit
def _run(actions_idx, val):
    idx2 = jnp.broadcast_to(actions_idx.astype(jnp.int32)[None, :], (8, _B))
    val2 = jnp.broadcast_to(val.astype(jnp.float32)[None, :], (8, _B))
    out_t = pl.pallas_call(
        _fill_body,
        grid=(_V // _JB,),
        in_specs=[
            pl.BlockSpec((8, _B), lambda i: (0, 0)),
            pl.BlockSpec((8, _B), lambda i: (0, 0)),
        ],
        out_specs=pl.BlockSpec((_JB, _B), lambda i: (i, 0)),
        out_shape=jax.ShapeDtypeStruct((_V, _B), jnp.float32),
    )(idx2, val2)
    return out_t.T


def kernel(actions_idx, action_space_dim, val):
    del action_space_dim
    return _run(actions_idx, val)
